# Initial kernel scaffold; baseline (speedup 1.0000x reference)
import jax
import jax.numpy as jnp
from jax.experimental import pallas as pl


def kernel(pos_v, pos_u, neg_v, weights, table):
    raise NotImplementedError("write your pallas kernel here")


# speedup vs baseline: 5.5420x; 5.5420x over previous
import jax
import jax.numpy as jnp
from jax import lax
from jax.experimental import pallas as pl
from jax.experimental.pallas import tpu as pltpu
import jax.experimental.pallas.tpu_sc as plsc

B = 16384
D = 64
K = 20
NC, NS, L = 2, 16, 16
NW = NC * NS
BW = B // NW
C = 64
NCH = BW // C
IDXN_ROWS = C * K // 128


def _sc_body(pos_v_hbm, pos_u_hbm, neg_v_hbm, w_hbm, table_hbm,
             pos_out_hbm, neg_out_hbm, wexp_out_hbm,
             idxv, idxu, idxn, wbuf,
             rows_v, rows_u, rows_n,
             pos_o, neg_o, wexp_o, sem):
    wid = lax.axis_index("s") * NC + lax.axis_index("c")
    iota = lax.iota(jnp.int32, L)

    def chunk_body(c, carry):
        base = wid * BW + c * C
        pltpu.sync_copy(pos_v_hbm.at[pl.ds(base, C)], idxv)
        pltpu.sync_copy(pos_u_hbm.at[pl.ds(base, C)], idxu)
        pltpu.sync_copy(neg_v_hbm.at[pl.ds(base * K, C * K)], idxn)
        pltpu.sync_copy(w_hbm.at[pl.ds(base, C)], wbuf)
        cp_v = pltpu.async_copy(table_hbm.at[idxv], rows_v, sem)
        cp_u = pltpu.async_copy(table_hbm.at[idxu], rows_u, sem)
        cps = []
        for r in range(IDXN_ROWS):
            cps.append(pltpu.async_copy(
                table_hbm.at[idxn.at[pl.ds(r * 128, 128)]],
                rows_n.at[pl.ds(r * 128, 128)], sem))
        cp_v.wait()
        cp_u.wait()
        for cp in cps:
            cp.wait()

        def group_body(g, gcarry):
            gb = g * L
            row = gb + iota
            nrow0 = row * K
            wvec = wbuf[pl.ds(gb, L)]
            pos_acc = jnp.zeros((L,), jnp.float32)
            nacc = [jnp.zeros((L,), jnp.float32) for _ in range(K)]
            for j in range(D):
                colj = jnp.full((L,), j, jnp.int32)
                vj = plsc.load_gather(rows_v, [row, colj])
                uj = plsc.load_gather(rows_u, [row, colj])
                pos_acc = pos_acc + vj * uj
                for k in range(K):
                    nj = plsc.load_gather(rows_n, [nrow0 + k, colj])
                    nacc[k] = nacc[k] + nj * vj
            pos_o[pl.ds(gb, L)] = pos_acc
            for k in range(K):
                plsc.store_scatter(neg_o, [nrow0 + k], nacc[k])
                plsc.store_scatter(wexp_o, [nrow0 + k], wvec)
            return gcarry

        lax.fori_loop(0, C // L, group_body, 0)
        pltpu.sync_copy(pos_o, pos_out_hbm.at[pl.ds(base, C)])
        pltpu.sync_copy(neg_o, neg_out_hbm.at[pl.ds(base * K, C * K)])
        pltpu.sync_copy(wexp_o, wexp_out_hbm.at[pl.ds(base * K, C * K)])
        return carry

    lax.fori_loop(0, NCH, chunk_body, 0)


def _sc_scores(pos_v, pos_u, neg_v2d, weights, table):
    mesh = plsc.VectorSubcoreMesh(core_axis_name="c", subcore_axis_name="s")
    f = pl.kernel(
        _sc_body,
        mesh=mesh,
        out_type=[
            jax.ShapeDtypeStruct((B,), jnp.float32),
            jax.ShapeDtypeStruct((B * K,), jnp.float32),
            jax.ShapeDtypeStruct((B * K,), jnp.float32),
        ],
        scratch_types=[
            pltpu.VMEM((C,), jnp.int32),
            pltpu.VMEM((C,), jnp.int32),
            pltpu.VMEM((C * K,), jnp.int32),
            pltpu.VMEM((C,), jnp.float32),
            pltpu.VMEM((C, D), jnp.float32),
            pltpu.VMEM((C, D), jnp.float32),
            pltpu.VMEM((C * K, D), jnp.float32),
            pltpu.VMEM((C,), jnp.float32),
            pltpu.VMEM((C * K,), jnp.float32),
            pltpu.VMEM((C * K,), jnp.float32),
            pltpu.SemaphoreType.DMA,
        ],
        compiler_params=pltpu.CompilerParams(
            needs_layout_passes=False, use_tc_tiling_on_sc=False),
    )
    return f(pos_v, pos_u, neg_v2d, weights, table)


def _log_sigmoid(x):
    return jnp.minimum(x, 0.0) - jnp.log1p(jnp.exp(-jnp.abs(x)))


def _tc_body(pos_ref, w_ref, neg_ref, wexp_ref, out_ref):
    g = pl.program_id(0)
    part = jnp.sum(wexp_ref[...] * _log_sigmoid(-neg_ref[...]))

    @pl.when(g == 0)
    def _():
        out_ref[...] = jnp.reshape(
            -jnp.sum(w_ref[...] * _log_sigmoid(pos_ref[...])), (1, 1))

    out_ref[...] = out_ref[...] - jnp.reshape(part, (1, 1))


def kernel(pos_v, pos_u, neg_v, weights, table):
    pos_v = pos_v.astype(jnp.int32)
    pos_u = pos_u.astype(jnp.int32)
    negflat = neg_v.astype(jnp.int32).reshape(B * K)
    pos_s, neg_s, wexp = _sc_scores(pos_v, pos_u, negflat, weights, table)
    nrows = B * K // 128
    res = pl.pallas_call(
        _tc_body,
        grid=(B * K // (128 * 128),),
        in_specs=[
            pl.BlockSpec((B // 128, 128), lambda g: (0, 0)),
            pl.BlockSpec((B // 128, 128), lambda g: (0, 0)),
            pl.BlockSpec((128, 128), lambda g: (g, 0)),
            pl.BlockSpec((128, 128), lambda g: (g, 0)),
        ],
        out_specs=pl.BlockSpec((1, 1), lambda g: (0, 0)),
        out_shape=jax.ShapeDtypeStruct((1, 1), jnp.float32),
    )(pos_s.reshape(B // 128, 128), weights.reshape(B // 128, 128),
      neg_s.reshape(nrows, 128), wexp.reshape(nrows, 128))
    return res[0, 0]


# speedup vs baseline: 7.0426x; 1.2708x over previous
import jax
import jax.numpy as jnp
from jax import lax
from jax.experimental import pallas as pl
from jax.experimental.pallas import tpu as pltpu
import jax.experimental.pallas.tpu_sc as plsc

B = 16384
D = 64
K = 20
NC, NS, L = 2, 16, 16
NW = NC * NS
BW = B // NW
C = 64
NCH = BW // C
IDXN_ROWS = C * K // 128


def _sc_body(pos_v_hbm, pos_u_hbm, neg_v_hbm, w_hbm, table_hbm,
             pos_out_hbm, neg_out_hbm, wexp_out_hbm,
             idxv, idxu, idxn, wbuf,
             rows_v, rows_u, rows_n,
             pos_o, neg_o, wexp_o, sem):
    wid = lax.axis_index("s") * NC + lax.axis_index("c")
    iota = lax.iota(jnp.int32, L)

    def chunk_body(c, carry):
        base = wid * BW + c * C
        pltpu.sync_copy(pos_v_hbm.at[pl.ds(base, C)], idxv)
        pltpu.sync_copy(pos_u_hbm.at[pl.ds(base, C)], idxu)
        pltpu.sync_copy(neg_v_hbm.at[pl.ds(base * K, C * K)], idxn)
        pltpu.sync_copy(w_hbm.at[pl.ds(base, C)], wbuf)
        cp_v = pltpu.async_copy(table_hbm.at[idxv], rows_v, sem)
        cp_u = pltpu.async_copy(table_hbm.at[idxu], rows_u, sem)
        cps = []
        for r in range(IDXN_ROWS):
            cps.append(pltpu.async_copy(
                table_hbm.at[idxn.at[pl.ds(r * 128, 128)]],
                rows_n.at[pl.ds(r * 128, 128)], sem))
        cp_v.wait()
        cp_u.wait()
        for cp in cps:
            cp.wait()

        def group_body(g, gcarry):
            gb = g * L
            row = gb + iota
            nrow0 = row * K
            wvec = wbuf[pl.ds(gb, L)]
            pos_acc = jnp.zeros((L,), jnp.float32)
            nacc = [jnp.zeros((L,), jnp.float32) for _ in range(K)]
            for j in range(D):
                colj = (iota + j) & (D - 1)
                vj = plsc.load_gather(rows_v, [row, colj])
                uj = plsc.load_gather(rows_u, [row, colj])
                pos_acc = pos_acc + vj * uj
                for k in range(K):
                    nj = plsc.load_gather(rows_n, [nrow0 + k, colj])
                    nacc[k] = nacc[k] + nj * vj
            pos_o[pl.ds(gb, L)] = pos_acc
            for k in range(K):
                plsc.store_scatter(neg_o, [nrow0 + k], nacc[k])
                plsc.store_scatter(wexp_o, [nrow0 + k], wvec)
            return gcarry

        lax.fori_loop(0, C // L, group_body, 0)
        pltpu.sync_copy(pos_o, pos_out_hbm.at[pl.ds(base, C)])
        pltpu.sync_copy(neg_o, neg_out_hbm.at[pl.ds(base * K, C * K)])
        pltpu.sync_copy(wexp_o, wexp_out_hbm.at[pl.ds(base * K, C * K)])
        return carry

    lax.fori_loop(0, NCH, chunk_body, 0)


def _sc_scores(pos_v, pos_u, neg_v2d, weights, table):
    mesh = plsc.VectorSubcoreMesh(core_axis_name="c", subcore_axis_name="s")
    f = pl.kernel(
        _sc_body,
        mesh=mesh,
        out_type=[
            jax.ShapeDtypeStruct((B,), jnp.float32),
            jax.ShapeDtypeStruct((B * K,), jnp.float32),
            jax.ShapeDtypeStruct((B * K,), jnp.float32),
        ],
        scratch_types=[
            pltpu.VMEM((C,), jnp.int32),
            pltpu.VMEM((C,), jnp.int32),
            pltpu.VMEM((C * K,), jnp.int32),
            pltpu.VMEM((C,), jnp.float32),
            pltpu.VMEM((C, D), jnp.float32),
            pltpu.VMEM((C, D), jnp.float32),
            pltpu.VMEM((C * K, D), jnp.float32),
            pltpu.VMEM((C,), jnp.float32),
            pltpu.VMEM((C * K,), jnp.float32),
            pltpu.VMEM((C * K,), jnp.float32),
            pltpu.SemaphoreType.DMA,
        ],
        compiler_params=pltpu.CompilerParams(
            needs_layout_passes=False, use_tc_tiling_on_sc=False),
    )
    return f(pos_v, pos_u, neg_v2d, weights, table)


def _log_sigmoid(x):
    return jnp.minimum(x, 0.0) - jnp.log1p(jnp.exp(-jnp.abs(x)))


def _tc_body(pos_ref, w_ref, neg_ref, wexp_ref, out_ref):
    g = pl.program_id(0)
    part = jnp.sum(wexp_ref[...] * _log_sigmoid(-neg_ref[...]))

    @pl.when(g == 0)
    def _():
        out_ref[...] = jnp.reshape(
            -jnp.sum(w_ref[...] * _log_sigmoid(pos_ref[...])), (1, 1))

    out_ref[...] = out_ref[...] - jnp.reshape(part, (1, 1))


def kernel(pos_v, pos_u, neg_v, weights, table):
    pos_v = pos_v.astype(jnp.int32)
    pos_u = pos_u.astype(jnp.int32)
    negflat = neg_v.astype(jnp.int32).reshape(B * K)
    pos_s, neg_s, wexp = _sc_scores(pos_v, pos_u, negflat, weights, table)
    nrows = B * K // 128
    res = pl.pallas_call(
        _tc_body,
        grid=(B * K // (128 * 128),),
        in_specs=[
            pl.BlockSpec((B // 128, 128), lambda g: (0, 0)),
            pl.BlockSpec((B // 128, 128), lambda g: (0, 0)),
            pl.BlockSpec((128, 128), lambda g: (g, 0)),
            pl.BlockSpec((128, 128), lambda g: (g, 0)),
        ],
        out_specs=pl.BlockSpec((1, 1), lambda g: (0, 0)),
        out_shape=jax.ShapeDtypeStruct((1, 1), jnp.float32),
    )(pos_s.reshape(B // 128, 128), weights.reshape(B // 128, 128),
      neg_s.reshape(nrows, 128), wexp.reshape(nrows, 128))
    return res[0, 0]


# speedup vs baseline: 8.1176x; 1.1526x over previous
import jax
import jax.numpy as jnp
from jax import lax
from jax.experimental import pallas as pl
from jax.experimental.pallas import tpu as pltpu
import jax.experimental.pallas.tpu_sc as plsc

B = 16384
D = 64
K = 20
NC, NS, L = 2, 16, 16
NW = NC * NS
BW = B // NW
C = 32
NCH = BW // C
NGR = C * K // 128


def _sc_body(pos_v_hbm, pos_u_hbm, neg_v_hbm, w_hbm, table_hbm,
             pos_out_hbm, neg_out_hbm, wexp_out_hbm,
             idxv, idxu, idxn, wbuf,
             rows_v, rows_u, rows_n,
             pos_o, neg_o, wexp_o, sem0, sem1):
    wid = lax.axis_index("s") * NC + lax.axis_index("c")
    iota = lax.iota(jnp.int32, L)
    sems = (sem0, sem1)

    def fire(c, p):
        base = wid * BW + c * C
        pltpu.sync_copy(pos_v_hbm.at[pl.ds(base, C)], idxv.at[p])
        pltpu.sync_copy(pos_u_hbm.at[pl.ds(base, C)], idxu.at[p])
        pltpu.sync_copy(neg_v_hbm.at[pl.ds(base * K, C * K)], idxn.at[p])
        pltpu.sync_copy(w_hbm.at[pl.ds(base, C)], wbuf.at[p])
        pltpu.async_copy(table_hbm.at[idxv.at[p]], rows_v.at[p], sems[p])
        pltpu.async_copy(table_hbm.at[idxu.at[p]], rows_u.at[p], sems[p])
        for r in range(NGR):
            pltpu.async_copy(
                table_hbm.at[idxn.at[p].at[pl.ds(r * 128, 128)]],
                rows_n.at[p].at[pl.ds(r * 128, 128)], sems[p])

    def drain(p):
        pltpu.make_async_copy(
            table_hbm.at[pl.ds(0, C)], rows_v.at[p], sems[p]).wait()
        pltpu.make_async_copy(
            table_hbm.at[pl.ds(0, C)], rows_u.at[p], sems[p]).wait()
        pltpu.make_async_copy(
            table_hbm.at[pl.ds(0, C * K)], rows_n.at[p], sems[p]).wait()

    def compute(c, p):
        base = wid * BW + c * C
        rv, ru, rn = rows_v.at[p], rows_u.at[p], rows_n.at[p]
        wb = wbuf.at[p]

        JU = 4

        def group_body(g, gcarry):
            gb = g * L
            row = gb + iota
            nrow0 = row * K

            def j_body(jj, accs):
                pos_acc, nacc = accs
                nacc = list(nacc)
                for jx in range(JU):
                    colj = (iota + (jj * JU + jx)) & (D - 1)
                    vj = plsc.load_gather(rv, [row, colj])
                    uj = plsc.load_gather(ru, [row, colj])
                    pos_acc = pos_acc + vj * uj
                    for k in range(K):
                        nj = plsc.load_gather(rn, [nrow0 + k, colj])
                        nacc[k] = nacc[k] + nj * vj
                return pos_acc, tuple(nacc)

            pos_acc, nacc = lax.fori_loop(
                0, D // JU, j_body,
                (jnp.zeros((L,), jnp.float32),
                 tuple(jnp.zeros((L,), jnp.float32) for _ in range(K))))
            wvec = wb[pl.ds(gb, L)]
            pos_o[pl.ds(gb, L)] = pos_acc
            for k in range(K):
                plsc.store_scatter(neg_o, [nrow0 + k], nacc[k])
                plsc.store_scatter(wexp_o, [nrow0 + k], wvec)
            return gcarry

        lax.fori_loop(0, C // L, group_body, 0)
        pltpu.sync_copy(pos_o, pos_out_hbm.at[pl.ds(base, C)])
        pltpu.sync_copy(neg_o, neg_out_hbm.at[pl.ds(base * K, C * K)])
        pltpu.sync_copy(wexp_o, wexp_out_hbm.at[pl.ds(base * K, C * K)])

    fire(0, 0)

    def outer(cc, carry):
        c0 = cc * 2
        fire(c0 + 1, 1)
        drain(0)
        compute(c0, 0)

        @pl.when(c0 + 2 < NCH)
        def _():
            fire(c0 + 2, 0)

        drain(1)
        compute(c0 + 1, 1)
        return carry

    lax.fori_loop(0, NCH // 2, outer, 0)


def _sc_scores(pos_v, pos_u, neg_v2d, weights, table):
    mesh = plsc.VectorSubcoreMesh(core_axis_name="c", subcore_axis_name="s")
    f = pl.kernel(
        _sc_body,
        mesh=mesh,
        out_type=[
            jax.ShapeDtypeStruct((B,), jnp.float32),
            jax.ShapeDtypeStruct((B * K,), jnp.float32),
            jax.ShapeDtypeStruct((B * K,), jnp.float32),
        ],
        scratch_types=[
            pltpu.VMEM((2, C), jnp.int32),
            pltpu.VMEM((2, C), jnp.int32),
            pltpu.VMEM((2, C * K), jnp.int32),
            pltpu.VMEM((2, C), jnp.float32),
            pltpu.VMEM((2, C, D), jnp.float32),
            pltpu.VMEM((2, C, D), jnp.float32),
            pltpu.VMEM((2, C * K, D), jnp.float32),
            pltpu.VMEM((C,), jnp.float32),
            pltpu.VMEM((C * K,), jnp.float32),
            pltpu.VMEM((C * K,), jnp.float32),
            pltpu.SemaphoreType.DMA,
            pltpu.SemaphoreType.DMA,
        ],
        compiler_params=pltpu.CompilerParams(
            needs_layout_passes=False, use_tc_tiling_on_sc=False),
    )
    return f(pos_v, pos_u, neg_v2d, weights, table)


def _log_sigmoid(x):
    return jnp.minimum(x, 0.0) - jnp.log1p(jnp.exp(-jnp.abs(x)))


def _tc_body(pos_ref, w_ref, neg_ref, wexp_ref, out_ref):
    g = pl.program_id(0)
    part = jnp.sum(wexp_ref[...] * _log_sigmoid(-neg_ref[...]))

    @pl.when(g == 0)
    def _():
        out_ref[...] = jnp.reshape(
            -jnp.sum(w_ref[...] * _log_sigmoid(pos_ref[...])), (1, 1))

    out_ref[...] = out_ref[...] - jnp.reshape(part, (1, 1))


def kernel(pos_v, pos_u, neg_v, weights, table):
    pos_v = pos_v.astype(jnp.int32)
    pos_u = pos_u.astype(jnp.int32)
    negflat = neg_v.astype(jnp.int32).reshape(B * K)
    pos_s, neg_s, wexp = _sc_scores(pos_v, pos_u, negflat, weights, table)
    nrows = B * K // 128
    res = pl.pallas_call(
        _tc_body,
        grid=(B * K // (128 * 128),),
        in_specs=[
            pl.BlockSpec((B // 128, 128), lambda g: (0, 0)),
            pl.BlockSpec((B // 128, 128), lambda g: (0, 0)),
            pl.BlockSpec((128, 128), lambda g: (g, 0)),
            pl.BlockSpec((128, 128), lambda g: (g, 0)),
        ],
        out_specs=pl.BlockSpec((1, 1), lambda g: (0, 0)),
        out_shape=jax.ShapeDtypeStruct((1, 1), jnp.float32),
    )(pos_s.reshape(B // 128, 128), weights.reshape(B // 128, 128),
      neg_s.reshape(nrows, 128), wexp.reshape(nrows, 128))
    return res[0, 0]
